# Initial kernel scaffold; baseline (speedup 1.0000x reference)
import functools

import jax
import jax.numpy as jnp
from jax import lax
from jax.experimental import pallas as pl
from jax.experimental.pallas import tpu as pltpu
from jax.experimental.pallas import tpu_sc as plsc

_NC = 2
_NS = 16
_LANES = 16

_K = 80
_ZROWS = 125


def _sc_agg(N, D, E):
    n_tiles = _NC * _NS
    edges_per_tile = E // n_tiles
    chunks = edges_per_tile // _K
    rows_per_tile = N // _NS
    n_zcopy = rows_per_tile // _ZROWS
    assert edges_per_tile * n_tiles == E
    assert chunks * _K == edges_per_tile
    assert n_zcopy * _ZROWS == rows_per_tile and rows_per_tile * _NS == N
    assert D % _LANES == 0

    mesh = plsc.VectorSubcoreMesh(core_axis_name="c", subcore_axis_name="s")

    @functools.partial(
        pl.kernel,
        mesh=mesh,
        out_type=jax.ShapeDtypeStruct((_NC, N, D), jnp.float32),
        scratch_types=[
            pltpu.VMEM((chunks, _K), jnp.int32),
            pltpu.VMEM((chunks, _K), jnp.int32),
            pltpu.VMEM((chunks, _K), jnp.float32),
            pltpu.VMEM((_K,), jnp.int32),
            pltpu.VMEM((_K, D), jnp.float32),
            pltpu.VMEM((_ZROWS, D), jnp.float32),
            pltpu.VMEM_SHARED((N, D), jnp.float32),
            pltpu.SemaphoreType.DMA,
        ],
    )
    def k(src_hbm, dst_hbm, w_hbm, x_hbm, out_hbm,
          src_v, dst_v, w_v, dstb, rows, zbuf, agg, sem):
        c = lax.axis_index("c")
        s = lax.axis_index("s")
        t = c * _NS + s

        pltpu.sync_copy(src_hbm.at[pl.ds(t * chunks, chunks)], src_v)
        pltpu.sync_copy(dst_hbm.at[pl.ds(t * chunks, chunks)], dst_v)
        pltpu.sync_copy(w_hbm.at[pl.ds(t * chunks, chunks)], w_v)

        def zrow(r, carry):
            for g in range(D // _LANES):
                zbuf[r, pl.ds(g * _LANES, _LANES)] = jnp.zeros(
                    (_LANES,), jnp.float32)
            return carry
        lax.fori_loop(0, _ZROWS, zrow, 0)
        for j in range(n_zcopy):
            pltpu.sync_copy(
                zbuf, agg.at[pl.ds(s * rows_per_tile + j * _ZROWS, _ZROWS)])
        plsc.subcore_barrier()

        def chunk_body(i, carry):
            for g in range(_K // _LANES):
                sl = pl.ds(g * _LANES, _LANES)
                dstb[sl] = dst_v[i, sl]
            pltpu.async_copy(x_hbm.at[src_v.at[i]], rows, sem).wait()

            def scale(e, carry2):
                wb = plsc.load_gather(
                    w_v, [jnp.full((_LANES,), i, jnp.int32),
                          jnp.full((_LANES,), e, jnp.int32)])
                for g in range(D // _LANES):
                    sl = pl.ds(g * _LANES, _LANES)
                    rows[e, sl] = rows[e, sl] * wb
                return carry2
            lax.fori_loop(0, _K, scale, 0)

            pltpu.sync_copy(rows, agg.at[dstb], add=True)
            return carry
        lax.fori_loop(0, chunks, chunk_body, 0)

        plsc.subcore_barrier()
        for j in range(n_zcopy):
            sl = pl.ds(s * rows_per_tile + j * _ZROWS, _ZROWS)
            pltpu.sync_copy(agg.at[sl], zbuf)
            pltpu.sync_copy(zbuf, out_hbm.at[c, sl])

    return k


def _dense(N, D, R):
    assert N % R == 0

    def body(p_ref, x_ref, wrel_ref, wroot_ref, wlin_ref,
             brel_ref, blin_ref, out_ref):
        dn = (((1,), (1,)), ((), ()))
        agg = p_ref[0] + p_ref[1]
        h = lax.dot_general(agg, wrel_ref[...], dn,
                            precision=lax.Precision.HIGHEST)
        h = h + lax.dot_general(x_ref[...], wroot_ref[...], dn,
                                precision=lax.Precision.HIGHEST)
        h = h + brel_ref[...]
        h = jnp.where(h >= 0, h, 0.01 * h)
        o = lax.dot_general(h, wlin_ref[...], dn,
                            precision=lax.Precision.HIGHEST)
        out_ref[...] = o + blin_ref[...]

    return pl.pallas_call(
        body,
        grid=(N // R,),
        in_specs=[
            pl.BlockSpec((2, R, D), lambda i: (0, i, 0)),
            pl.BlockSpec((R, D), lambda i: (i, 0)),
            pl.BlockSpec((D, D), lambda i: (0, 0)),
            pl.BlockSpec((D, D), lambda i: (0, 0)),
            pl.BlockSpec((D, D), lambda i: (0, 0)),
            pl.BlockSpec((1, D), lambda i: (0, 0)),
            pl.BlockSpec((1, D), lambda i: (0, 0)),
        ],
        out_specs=pl.BlockSpec((R, D), lambda i: (i, 0)),
        out_shape=jax.ShapeDtypeStruct((N, D), jnp.float32),
    )


def kernel(x, edge_index, edge_attr, W_rel, b_rel, W_root, W_lin, b_lin):
    N, D = x.shape
    E = edge_index.shape[1]
    src2 = edge_index[0].reshape(E // _K, _K)
    dst2 = edge_index[1].reshape(E // _K, _K)
    w2 = edge_attr.reshape(E // _K, _K)
    partial = _sc_agg(N, D, E)(src2, dst2, w2, x)
    return _dense(N, D, 400)(partial, x, W_rel, W_root, W_lin,
                             b_rel.reshape(1, D), b_lin.reshape(1, D))


# speedup vs baseline: 3.9957x; 3.9957x over previous
import functools

import jax
import jax.numpy as jnp
from jax import lax
from jax.experimental import pallas as pl
from jax.experimental.pallas import tpu as pltpu
from jax.experimental.pallas import tpu_sc as plsc

_NC = 2
_NS = 16
_LANES = 16

_K = 80
_ZROWS = 128


def _sc_agg(N, D, E):
    H = D // _NC
    edges_per_tile = E // _NS
    chunks = edges_per_tile // _K
    npad = ((N + _NS * _ZROWS - 1) // (_NS * _ZROWS)) * _NS * _ZROWS
    rows_per_tile = npad // _NS
    n_zcopy = rows_per_tile // _ZROWS
    assert edges_per_tile * _NS == E
    assert chunks * _K == edges_per_tile
    assert H % _LANES == 0 and _K % _LANES == 0

    mesh = plsc.VectorSubcoreMesh(core_axis_name="c", subcore_axis_name="s")

    @functools.partial(
        pl.kernel,
        mesh=mesh,
        compiler_params=pltpu.CompilerParams(use_tc_tiling_on_sc=False),
        out_type=jax.ShapeDtypeStruct((_NC, npad, H), jnp.float32),
        scratch_types=[
            pltpu.VMEM((edges_per_tile,), jnp.int32),
            pltpu.VMEM((edges_per_tile,), jnp.int32),
            pltpu.VMEM((edges_per_tile,), jnp.float32),
            pltpu.VMEM((_K,), jnp.int32),
            pltpu.VMEM((_K,), jnp.int32),
            pltpu.VMEM((_K, H), jnp.float32),
            pltpu.VMEM((_ZROWS, H), jnp.float32),
            pltpu.VMEM_SHARED((npad, H), jnp.float32),
            pltpu.SemaphoreType.DMA,
        ],
    )
    def k(src_hbm, dst_hbm, w_hbm, x2_hbm, out_hbm,
          src_v, dst_v, w_v, srcb, dstb, rows, zbuf, agg, sem):
        c = lax.axis_index("c")
        s = lax.axis_index("s")

        eb = s * edges_per_tile
        pltpu.sync_copy(src_hbm.at[pl.ds(eb, edges_per_tile)], src_v)
        pltpu.sync_copy(dst_hbm.at[pl.ds(eb, edges_per_tile)], dst_v)
        pltpu.sync_copy(w_hbm.at[pl.ds(eb, edges_per_tile)], w_v)

        def zrow(r, carry):
            for g in range(H // _LANES):
                zbuf[r, pl.ds(g * _LANES, _LANES)] = jnp.zeros(
                    (_LANES,), jnp.float32)
            return carry
        lax.fori_loop(0, _ZROWS, zrow, 0)
        for j in range(n_zcopy):
            pltpu.sync_copy(
                zbuf, agg.at[pl.ds(s * rows_per_tile + j * _ZROWS, _ZROWS)])
        plsc.subcore_barrier()

        cshift = jnp.full((_LANES,), c * N, jnp.int32)

        def chunk_body(i, carry):
            for g in range(_K // _LANES):
                sl = pl.ds(g * _LANES, _LANES)
                esl = pl.ds(i * _K + g * _LANES, _LANES)
                srcb[sl] = src_v[esl] + cshift
                dstb[sl] = dst_v[esl]
            pltpu.async_copy(x2_hbm.at[srcb], rows, sem).wait()

            def scale(g, carry2):
                w16 = w_v[pl.ds(i * _K + g * _LANES, _LANES)]
                for j in range(_LANES):
                    e = g * _LANES + j
                    wb = jnp.full((_LANES,), w16[j], jnp.float32)
                    for f in range(H // _LANES):
                        sl = pl.ds(f * _LANES, _LANES)
                        rows[e, sl] = rows[e, sl] * wb
                return carry2
            lax.fori_loop(0, _K // _LANES, scale, 0)

            pltpu.sync_copy(rows, agg.at[dstb], add=True)
            return carry
        lax.fori_loop(0, chunks, chunk_body, 0)

        plsc.subcore_barrier()
        for j in range(n_zcopy):
            sl = pl.ds(s * rows_per_tile + j * _ZROWS, _ZROWS)
            pltpu.sync_copy(agg.at[sl], zbuf)
            pltpu.sync_copy(zbuf, out_hbm.at[c, sl])

    return k


def _dense(N, D, R):
    assert N % R == 0

    def body(p_ref, x_ref, wrel_ref, wroot_ref, wlin_ref,
             brel_ref, blin_ref, out_ref):
        dn = (((1,), (1,)), ((), ()))
        agg = jnp.concatenate([p_ref[0], p_ref[1]], axis=1)
        h = lax.dot_general(agg, wrel_ref[...], dn,
                            precision=lax.Precision.HIGHEST)
        h = h + lax.dot_general(x_ref[...], wroot_ref[...], dn,
                                precision=lax.Precision.HIGHEST)
        h = h + brel_ref[...]
        h = jnp.where(h >= 0, h, 0.01 * h)
        o = lax.dot_general(h, wlin_ref[...], dn,
                            precision=lax.Precision.HIGHEST)
        out_ref[...] = o + blin_ref[...]

    return pl.pallas_call(
        body,
        grid=(N // R,),
        in_specs=[
            pl.BlockSpec((2, R, D // _NC), lambda i: (0, i, 0)),
            pl.BlockSpec((R, D), lambda i: (i, 0)),
            pl.BlockSpec((D, D), lambda i: (0, 0)),
            pl.BlockSpec((D, D), lambda i: (0, 0)),
            pl.BlockSpec((D, D), lambda i: (0, 0)),
            pl.BlockSpec((1, D), lambda i: (0, 0)),
            pl.BlockSpec((1, D), lambda i: (0, 0)),
        ],
        out_specs=pl.BlockSpec((R, D), lambda i: (i, 0)),
        out_shape=jax.ShapeDtypeStruct((N, D), jnp.float32),
    )


def kernel(x, edge_index, edge_attr, W_rel, b_rel, W_root, W_lin, b_lin):
    N, D = x.shape
    E = edge_index.shape[1]
    H = D // _NC
    x2 = jnp.concatenate([x[:, :H], x[:, H:]], axis=0)
    partial = _sc_agg(N, D, E)(edge_index[0], edge_index[1], edge_attr, x2)
    return _dense(N, D, 400)(partial, x, W_rel, W_root, W_lin,
                             b_rel.reshape(1, D), b_lin.reshape(1, D))


# speedup vs baseline: 6.6225x; 1.6574x over previous
import functools

import jax
import jax.numpy as jnp
from jax import lax
from jax.experimental import pallas as pl
from jax.experimental.pallas import tpu as pltpu
from jax.experimental.pallas import tpu_sc as plsc

_NC = 2
_NS = 16
_LANES = 16

_K = 80
_ZROWS = 64
_NBUF = 5


def _sc_agg(N, D, E):
    H = D // _NC
    edges_per_tile = E // _NS
    chunks = edges_per_tile // _K
    npad = ((N + _NS * _ZROWS - 1) // (_NS * _ZROWS)) * _NS * _ZROWS
    rows_per_tile = npad // _NS
    n_zcopy = rows_per_tile // _ZROWS
    assert edges_per_tile * _NS == E
    assert chunks * _K == edges_per_tile
    assert H % _LANES == 0 and _K % _LANES == 0
    assert chunks % _NBUF == 0 and chunks // _NBUF >= 3 and _NBUF == 5

    mesh = plsc.VectorSubcoreMesh(core_axis_name="c", subcore_axis_name="s")

    @functools.partial(
        pl.kernel,
        mesh=mesh,
        compiler_params=pltpu.CompilerParams(use_tc_tiling_on_sc=False),
        out_type=jax.ShapeDtypeStruct((_NC, npad, H), jnp.float32),
        scratch_types=[
            pltpu.VMEM((edges_per_tile,), jnp.int32),
            pltpu.VMEM((edges_per_tile,), jnp.int32),
            pltpu.VMEM((_NBUF, _K), jnp.float32),
            pltpu.VMEM((_NBUF, _K), jnp.int32),
            pltpu.VMEM((_NBUF, _K), jnp.int32),
            pltpu.VMEM((_NBUF, _K, H), jnp.float32),
            pltpu.VMEM((_ZROWS, H), jnp.float32),
            pltpu.VMEM_SHARED((npad, H), jnp.float32),
            pltpu.SemaphoreType.DMA,
            pltpu.SemaphoreType.DMA,
        ],
    )
    def k(src_hbm, dst_hbm, w_hbm, x2_hbm, out_hbm,
          src_v, dst_v, wring, srcb, dstb, rows, zbuf, agg, gsem, ssem):
        c = lax.axis_index("c")
        s = lax.axis_index("s")

        eb = s * edges_per_tile
        pltpu.sync_copy(src_hbm.at[pl.ds(eb, edges_per_tile)], src_v)
        pltpu.sync_copy(dst_hbm.at[pl.ds(eb, edges_per_tile)], dst_v)

        def zrow(r, carry):
            for g in range(H // _LANES):
                zbuf[r, pl.ds(g * _LANES, _LANES)] = jnp.zeros(
                    (_LANES,), jnp.float32)
            return carry
        lax.fori_loop(0, _ZROWS, zrow, 0)
        for j in range(n_zcopy):
            pltpu.sync_copy(
                zbuf, agg.at[pl.ds(s * rows_per_tile + j * _ZROWS, _ZROWS)])
        plsc.subcore_barrier()

        cshift = jnp.full((_LANES,), c * N, jnp.int32)

        def prep(i, b):
            pltpu.async_copy(
                w_hbm.at[pl.ds(eb + i * _K, _K)], wring.at[b], gsem)
            for g in range(_K // _LANES):
                sl = pl.ds(g * _LANES, _LANES)
                esl = pl.ds(i * _K + g * _LANES, _LANES)
                srcb[b, sl] = src_v[esl] + cshift
                dstb[b, sl] = dst_v[esl]
            pltpu.async_copy(x2_hbm.at[srcb.at[b]], rows.at[b], gsem)

        def scale_rows(b):
            @plsc.parallel_loop(0, _K // _LANES)
            def scale(g):
                w16 = wring[b, pl.ds(g * _LANES, _LANES)]
                for j in range(_LANES):
                    e = g * _LANES + j
                    wb = jnp.full((_LANES,), w16[j], jnp.float32)
                    for f in range(H // _LANES):
                        sl = pl.ds(f * _LANES, _LANES)
                        rows[b, e, sl] = rows[b, e, sl] * wb

        def work(i, b, wait_scatter, do_prep):
            pltpu.make_async_copy(
                w_hbm.at[pl.ds(eb + i * _K, _K)], wring.at[b], gsem).wait()
            pltpu.make_async_copy(
                x2_hbm.at[srcb.at[b]], rows.at[b], gsem).wait()
            scale_rows(b)
            pltpu.async_copy(rows.at[b], agg.at[dstb.at[b]], ssem, add=True)
            if wait_scatter:
                pltpu.make_async_copy(
                    rows.at[b], agg.at[dstb.at[b]], ssem).wait()
            if do_prep:
                prep(i + 2, (b + 2) % _NBUF)

        prep(0, 0)
        prep(1, 1)
        for b in range(_NBUF):
            work(b, b, wait_scatter=b >= 3, do_prep=True)

        def group(q, carry):
            for b in range(_NBUF):
                work(q * _NBUF + b, b, wait_scatter=True, do_prep=True)
            return carry
        lax.fori_loop(1, chunks // _NBUF - 1, group, 0)

        base = chunks - _NBUF
        for b in range(_NBUF):
            work(base + b, b, wait_scatter=True, do_prep=b < 3)
        for b in range(3):
            pltpu.make_async_copy(
                rows.at[b], agg.at[dstb.at[b]], ssem).wait()

        plsc.subcore_barrier()
        for j in range(n_zcopy):
            sl = pl.ds(s * rows_per_tile + j * _ZROWS, _ZROWS)
            pltpu.sync_copy(agg.at[sl], zbuf)
            pltpu.sync_copy(zbuf, out_hbm.at[c, sl])

    return k


def _dense(N, D, R):
    assert N % R == 0

    def body(p_ref, x_ref, wrel_ref, wroot_ref, wlin_ref,
             brel_ref, blin_ref, out_ref):
        dn = (((1,), (1,)), ((), ()))
        agg = jnp.concatenate([p_ref[0], p_ref[1]], axis=1)
        h = lax.dot_general(agg, wrel_ref[...], dn,
                            precision=lax.Precision.HIGHEST)
        h = h + lax.dot_general(x_ref[...], wroot_ref[...], dn,
                                precision=lax.Precision.HIGHEST)
        h = h + brel_ref[...]
        h = jnp.where(h >= 0, h, 0.01 * h)
        o = lax.dot_general(h, wlin_ref[...], dn,
                            precision=lax.Precision.HIGHEST)
        out_ref[...] = o + blin_ref[...]

    return pl.pallas_call(
        body,
        grid=(N // R,),
        in_specs=[
            pl.BlockSpec((2, R, D // _NC), lambda i: (0, i, 0)),
            pl.BlockSpec((R, D), lambda i: (i, 0)),
            pl.BlockSpec((D, D), lambda i: (0, 0)),
            pl.BlockSpec((D, D), lambda i: (0, 0)),
            pl.BlockSpec((D, D), lambda i: (0, 0)),
            pl.BlockSpec((1, D), lambda i: (0, 0)),
            pl.BlockSpec((1, D), lambda i: (0, 0)),
        ],
        out_specs=pl.BlockSpec((R, D), lambda i: (i, 0)),
        out_shape=jax.ShapeDtypeStruct((N, D), jnp.float32),
    )


def kernel(x, edge_index, edge_attr, W_rel, b_rel, W_root, W_lin, b_lin):
    N, D = x.shape
    E = edge_index.shape[1]
    H = D // _NC
    x2 = jnp.concatenate([x[:, :H], x[:, H:]], axis=0)
    partial = _sc_agg(N, D, E)(edge_index[0], edge_index[1], edge_attr, x2)
    return _dense(N, D, 400)(partial, x, W_rel, W_root, W_lin,
                             b_rel.reshape(1, D), b_lin.reshape(1, D))


# speedup vs baseline: 6.7072x; 1.0128x over previous
import functools

import jax
import jax.numpy as jnp
from jax import lax
from jax.experimental import pallas as pl
from jax.experimental.pallas import tpu as pltpu
from jax.experimental.pallas import tpu_sc as plsc

_NC = 2
_NS = 16
_LANES = 16

_K = 80
_ZROWS = 64
_NBUF = 5


def _sc_agg(N, D, E):
    H = D // _NC
    edges_per_tile = E // _NS
    chunks = edges_per_tile // _K
    npad = ((N + _NS * _ZROWS - 1) // (_NS * _ZROWS)) * _NS * _ZROWS
    rows_per_tile = npad // _NS
    n_zcopy = rows_per_tile // _ZROWS
    assert edges_per_tile * _NS == E
    assert chunks * _K == edges_per_tile
    assert H % _LANES == 0 and _K % _LANES == 0
    assert chunks % _NBUF == 0 and chunks // _NBUF >= 3 and _NBUF == 5

    mesh = plsc.VectorSubcoreMesh(core_axis_name="c", subcore_axis_name="s")

    @functools.partial(
        pl.kernel,
        mesh=mesh,
        compiler_params=pltpu.CompilerParams(use_tc_tiling_on_sc=False),
        out_type=jax.ShapeDtypeStruct((_NC, npad, H), jnp.float32),
        scratch_types=[
            pltpu.VMEM((edges_per_tile,), jnp.int32),
            pltpu.VMEM((edges_per_tile,), jnp.int32),
            pltpu.VMEM((_NBUF, _K), jnp.float32),
            pltpu.VMEM((_NBUF, _K), jnp.int32),
            pltpu.VMEM((_NBUF, _K), jnp.int32),
            pltpu.VMEM((_NBUF, _K, H), jnp.float32),
            pltpu.VMEM((_ZROWS, H), jnp.float32),
            pltpu.VMEM_SHARED((npad, H), jnp.float32),
            pltpu.SemaphoreType.DMA,
            pltpu.SemaphoreType.DMA,
        ],
    )
    def k(src_hbm, dst_hbm, w_hbm, x2_hbm, out_hbm,
          src_v, dst_v, wring, srcb, dstb, rows, zbuf, agg, gsem, ssem):
        c = lax.axis_index("c")
        s = lax.axis_index("s")

        eb = s * edges_per_tile
        pltpu.sync_copy(src_hbm.at[pl.ds(eb, edges_per_tile)], src_v)
        pltpu.sync_copy(dst_hbm.at[pl.ds(eb, edges_per_tile)], dst_v)

        cshift = jnp.full((_LANES,), c * N, jnp.int32)

        def prep(i, b):
            pltpu.async_copy(
                w_hbm.at[pl.ds(eb + i * _K, _K)], wring.at[b], gsem)
            for g in range(_K // _LANES):
                sl = pl.ds(g * _LANES, _LANES)
                esl = pl.ds(i * _K + g * _LANES, _LANES)
                srcb[b, sl] = src_v[esl] + cshift
                dstb[b, sl] = dst_v[esl]
            pltpu.async_copy(x2_hbm.at[srcb.at[b]], rows.at[b], gsem)

        def scale_rows(b):
            @plsc.parallel_loop(0, _K // _LANES)
            def scale(g):
                w16 = wring[b, pl.ds(g * _LANES, _LANES)]
                for j in range(_LANES):
                    e = g * _LANES + j
                    wb = jnp.full((_LANES,), w16[j], jnp.float32)
                    for f in range(H // _LANES):
                        sl = pl.ds(f * _LANES, _LANES)
                        rows[b, e, sl] = rows[b, e, sl] * wb

        def work(i, b, wait_scatter, do_prep):
            pltpu.make_async_copy(
                w_hbm.at[pl.ds(eb + i * _K, _K)], wring.at[b], gsem).wait()
            pltpu.make_async_copy(
                x2_hbm.at[srcb.at[b]], rows.at[b], gsem).wait()
            scale_rows(b)
            pltpu.async_copy(rows.at[b], agg.at[dstb.at[b]], ssem, add=True)
            if wait_scatter:
                pltpu.make_async_copy(
                    rows.at[b], agg.at[dstb.at[b]], ssem).wait()
            if do_prep:
                prep(i + 2, (b + 2) % _NBUF)

        prep(0, 0)
        prep(1, 1)

        def zrow(r, carry):
            for g in range(H // _LANES):
                zbuf[r, pl.ds(g * _LANES, _LANES)] = jnp.zeros(
                    (_LANES,), jnp.float32)
            return carry
        lax.fori_loop(0, _ZROWS, zrow, 0)
        for j in range(n_zcopy):
            pltpu.sync_copy(
                zbuf, agg.at[pl.ds(s * rows_per_tile + j * _ZROWS, _ZROWS)])
        plsc.subcore_barrier()

        for b in range(_NBUF):
            work(b, b, wait_scatter=b >= 3, do_prep=True)

        def group(q, carry):
            for b in range(_NBUF):
                work(q * _NBUF + b, b, wait_scatter=True, do_prep=True)
            return carry
        lax.fori_loop(1, chunks // _NBUF - 1, group, 0)

        base = chunks - _NBUF
        for b in range(_NBUF):
            work(base + b, b, wait_scatter=True, do_prep=b < 3)
        for b in range(3):
            pltpu.make_async_copy(
                rows.at[b], agg.at[dstb.at[b]], ssem).wait()

        plsc.subcore_barrier()
        sl = pl.ds(s * rows_per_tile, rows_per_tile)
        pltpu.sync_copy(agg.at[sl], out_hbm.at[c, sl])

    return k


def _dense_root(N, D, R):
    assert N % R == 0

    def body(x_ref, wroot_ref, brel_ref, out_ref):
        dn = (((1,), (1,)), ((), ()))
        out_ref[...] = lax.dot_general(
            x_ref[...], wroot_ref[...], dn,
            precision=lax.Precision.HIGHEST) + brel_ref[...]

    return pl.pallas_call(
        body,
        grid=(N // R,),
        in_specs=[
            pl.BlockSpec((R, D), lambda i: (i, 0)),
            pl.BlockSpec((D, D), lambda i: (0, 0)),
            pl.BlockSpec((1, D), lambda i: (0, 0)),
        ],
        out_specs=pl.BlockSpec((R, D), lambda i: (i, 0)),
        out_shape=jax.ShapeDtypeStruct((N, D), jnp.float32),
    )


def _dense_rest(N, D, R):
    assert N % R == 0

    def body(p_ref, r_ref, wrel_ref, wlin_ref, blin_ref, out_ref):
        dn = (((1,), (1,)), ((), ()))
        agg = jnp.concatenate([p_ref[0], p_ref[1]], axis=1)
        h = lax.dot_general(agg, wrel_ref[...], dn,
                            precision=lax.Precision.HIGHEST) + r_ref[...]
        h = jnp.where(h >= 0, h, 0.01 * h)
        o = lax.dot_general(h, wlin_ref[...], dn,
                            precision=lax.Precision.HIGHEST)
        out_ref[...] = o + blin_ref[...]

    return pl.pallas_call(
        body,
        grid=(N // R,),
        in_specs=[
            pl.BlockSpec((2, R, D // _NC), lambda i: (0, i, 0)),
            pl.BlockSpec((R, D), lambda i: (i, 0)),
            pl.BlockSpec((D, D), lambda i: (0, 0)),
            pl.BlockSpec((D, D), lambda i: (0, 0)),
            pl.BlockSpec((1, D), lambda i: (0, 0)),
        ],
        out_specs=pl.BlockSpec((R, D), lambda i: (i, 0)),
        out_shape=jax.ShapeDtypeStruct((N, D), jnp.float32),
    )


def kernel(x, edge_index, edge_attr, W_rel, b_rel, W_root, W_lin, b_lin):
    N, D = x.shape
    E = edge_index.shape[1]
    H = D // _NC
    x2 = jnp.concatenate([x[:, :H], x[:, H:]], axis=0)
    r = _dense_root(N, D, 400)(x, W_root, b_rel.reshape(1, D))
    partial = _sc_agg(N, D, E)(edge_index[0], edge_index[1], edge_attr, x2)
    return _dense_rest(N, D, 400)(partial, r, W_rel, W_lin,
                                  b_lin.reshape(1, D))


# speedup vs baseline: 7.8312x; 1.1676x over previous
import functools

import jax
import jax.numpy as jnp
from jax import lax
from jax.experimental import pallas as pl
from jax.experimental.pallas import tpu as pltpu
from jax.experimental.pallas import tpu_sc as plsc

_NC = 2
_NS = 16
_LANES = 16

_K = 80
_ZROWS = 64
_NBUF = 5


def _sc_agg(N, D, E):
    H = D // _NC
    edges_per_tile = E // _NS
    chunks = edges_per_tile // _K
    npad = ((N + _NS * _ZROWS - 1) // (_NS * _ZROWS)) * _NS * _ZROWS
    rows_per_tile = npad // _NS
    n_zcopy = rows_per_tile // _ZROWS
    assert edges_per_tile * _NS == E
    assert chunks * _K == edges_per_tile
    assert H % _LANES == 0 and _K % _LANES == 0
    assert chunks % _NBUF == 0 and chunks // _NBUF >= 3 and _NBUF == 5

    mesh = plsc.VectorSubcoreMesh(core_axis_name="c", subcore_axis_name="s")

    @functools.partial(
        pl.kernel,
        mesh=mesh,
        compiler_params=pltpu.CompilerParams(use_tc_tiling_on_sc=False,
                                             needs_layout_passes=False),
        out_type=jax.ShapeDtypeStruct((_NC, npad, H), jnp.bfloat16),
        scratch_types=[
            pltpu.VMEM((edges_per_tile,), jnp.int32),
            pltpu.VMEM((edges_per_tile,), jnp.int32),
            pltpu.VMEM((_NBUF, _K), jnp.float32),
            pltpu.VMEM((_NBUF, _K), jnp.int32),
            pltpu.VMEM((_NBUF, _K), jnp.int32),
            pltpu.VMEM((_NBUF, _K, H), jnp.bfloat16),
            pltpu.VMEM((_ZROWS, H), jnp.bfloat16),
            pltpu.VMEM_SHARED((npad, H), jnp.bfloat16),
            pltpu.SemaphoreType.DMA,
            pltpu.SemaphoreType.DMA,
        ],
    )
    def k(src_hbm, dst_hbm, w_hbm, x2_hbm, out_hbm,
          src_v, dst_v, wring, srcb, dstb, rows, zbuf, agg, gsem, ssem):
        c = lax.axis_index("c")
        s = lax.axis_index("s")

        eb = s * edges_per_tile
        pltpu.sync_copy(src_hbm.at[pl.ds(eb, edges_per_tile)], src_v)
        pltpu.sync_copy(dst_hbm.at[pl.ds(eb, edges_per_tile)], dst_v)

        cshift = jnp.full((_LANES,), c * N, jnp.int32)

        def prep(i, b):
            pltpu.async_copy(
                w_hbm.at[pl.ds(eb + i * _K, _K)], wring.at[b], gsem)
            for g in range(_K // _LANES):
                sl = pl.ds(g * _LANES, _LANES)
                esl = pl.ds(i * _K + g * _LANES, _LANES)
                srcb[b, sl] = src_v[esl] + cshift
                dstb[b, sl] = dst_v[esl]
            pltpu.async_copy(x2_hbm.at[srcb.at[b]], rows.at[b], gsem)

        def scale_rows(b):
            @plsc.parallel_loop(0, _K // _LANES)
            def scale(g):
                w16 = wring[b, pl.ds(g * _LANES, _LANES)]
                for j in range(_LANES):
                    e = g * _LANES + j
                    wb32 = jnp.full((_LANES,), w16[j], jnp.float32)
                    wb = plsc.pack(wb32, wb32,
                                   format=plsc.PackFormat.INTERLEAVED)
                    for f in range(H // (2 * _LANES)):
                        sl = pl.ds(f * 2 * _LANES, 2 * _LANES)
                        rows[b, e, sl] = rows[b, e, sl] * wb

        def work(i, b, wait_scatter, do_prep):
            pltpu.make_async_copy(
                w_hbm.at[pl.ds(eb + i * _K, _K)], wring.at[b], gsem).wait()
            pltpu.make_async_copy(
                x2_hbm.at[srcb.at[b]], rows.at[b], gsem).wait()
            scale_rows(b)
            pltpu.async_copy(rows.at[b], agg.at[dstb.at[b]], ssem, add=True)
            if wait_scatter:
                pltpu.make_async_copy(
                    rows.at[b], agg.at[dstb.at[b]], ssem).wait()
            if do_prep:
                prep(i + 2, (b + 2) % _NBUF)

        prep(0, 0)
        prep(1, 1)

        def zrow(r, carry):
            for g in range(H // (2 * _LANES)):
                zbuf[r, pl.ds(g * 2 * _LANES, 2 * _LANES)] = jnp.zeros(
                    (2 * _LANES,), jnp.bfloat16)
            return carry
        lax.fori_loop(0, _ZROWS, zrow, 0)
        for j in range(n_zcopy):
            pltpu.sync_copy(
                zbuf, agg.at[pl.ds(s * rows_per_tile + j * _ZROWS, _ZROWS)])
        plsc.subcore_barrier()

        for b in range(_NBUF):
            work(b, b, wait_scatter=b >= 3, do_prep=True)

        def group(q, carry):
            for b in range(_NBUF):
                work(q * _NBUF + b, b, wait_scatter=True, do_prep=True)
            return carry
        lax.fori_loop(1, chunks // _NBUF - 1, group, 0)

        base = chunks - _NBUF
        for b in range(_NBUF):
            work(base + b, b, wait_scatter=True, do_prep=b < 3)
        for b in range(3):
            pltpu.make_async_copy(
                rows.at[b], agg.at[dstb.at[b]], ssem).wait()

        plsc.subcore_barrier()
        sl = pl.ds(s * rows_per_tile, rows_per_tile)
        pltpu.sync_copy(agg.at[sl], out_hbm.at[c, sl])

    return k


def _dense_root(N, D, R):
    assert N % R == 0

    def body(x_ref, wroot_ref, brel_ref, out_ref):
        dn = (((1,), (1,)), ((), ()))
        out_ref[...] = lax.dot_general(
            x_ref[...], wroot_ref[...], dn,
            precision=lax.Precision.HIGHEST) + brel_ref[...]

    return pl.pallas_call(
        body,
        grid=(N // R,),
        in_specs=[
            pl.BlockSpec((R, D), lambda i: (i, 0)),
            pl.BlockSpec((D, D), lambda i: (0, 0)),
            pl.BlockSpec((1, D), lambda i: (0, 0)),
        ],
        out_specs=pl.BlockSpec((R, D), lambda i: (i, 0)),
        out_shape=jax.ShapeDtypeStruct((N, D), jnp.float32),
    )


def _dense_rest(N, D, R):
    assert N % R == 0

    def body(p_ref, r_ref, wrel_ref, wlin_ref, blin_ref, out_ref):
        dn = (((1,), (1,)), ((), ()))
        agg = jnp.concatenate(
            [p_ref[0], p_ref[1]], axis=1).astype(jnp.float32)
        h = lax.dot_general(agg, wrel_ref[...], dn,
                            precision=lax.Precision.HIGHEST) + r_ref[...]
        h = jnp.where(h >= 0, h, 0.01 * h)
        o = lax.dot_general(h, wlin_ref[...], dn,
                            precision=lax.Precision.HIGHEST)
        out_ref[...] = o + blin_ref[...]

    return pl.pallas_call(
        body,
        grid=(N // R,),
        in_specs=[
            pl.BlockSpec((2, R, D // _NC), lambda i: (0, i, 0)),
            pl.BlockSpec((R, D), lambda i: (i, 0)),
            pl.BlockSpec((D, D), lambda i: (0, 0)),
            pl.BlockSpec((D, D), lambda i: (0, 0)),
            pl.BlockSpec((1, D), lambda i: (0, 0)),
        ],
        out_specs=pl.BlockSpec((R, D), lambda i: (i, 0)),
        out_shape=jax.ShapeDtypeStruct((N, D), jnp.float32),
    )


def kernel(x, edge_index, edge_attr, W_rel, b_rel, W_root, W_lin, b_lin):
    N, D = x.shape
    E = edge_index.shape[1]
    H = D // _NC
    x2 = jnp.concatenate([x[:, :H], x[:, H:]],
                         axis=0).astype(jnp.bfloat16)
    r = _dense_root(N, D, 400)(x, W_root, b_rel.reshape(1, D))
    partial = _sc_agg(N, D, E)(edge_index[0], edge_index[1], edge_attr, x2)
    return _dense_rest(N, D, 400)(partial, r, W_rel, W_lin,
                                  b_lin.reshape(1, D))
